# baseline (speedup 1.0000x reference)
import jax
import jax.numpy as jnp
from jax.experimental import pallas as pl
from jax.experimental.pallas import tpu as pltpu

_TILE = 2048
_DCH = 384


def _moe_body(x_ref, wg_ref, wu_ref, wd_ref, o_ref, h_ref):
    j = pl.program_id(1)

    @pl.when(j == 0)
    def _prep():
        xt = x_ref[...]
        logits = jax.lax.dot_general(
            xt, wg_ref[...], (((1,), (1,)), ((), ())),
            preferred_element_type=jnp.float32)
        ne = logits.shape[-1]
        m1 = jnp.max(logits, axis=-1, keepdims=True)
        e = jnp.exp(logits - m1)
        denom = jnp.sum(e, axis=-1, keepdims=True)
        col = jax.lax.broadcasted_iota(jnp.int32, logits.shape, 1)
        is_max = logits == m1
        argmax1 = jnp.min(jnp.where(is_max, col, ne), axis=-1, keepdims=True)
        e2 = jnp.max(jnp.where(col == argmax1, 0.0, e), axis=-1, keepdims=True)
        p = (1.0 + e2) / denom
        scale = p / (p + 1e-9)
        up = jax.lax.dot_general(
            xt.astype(jnp.bfloat16), wu_ref[...].astype(jnp.bfloat16),
            (((1,), (1,)), ((), ())), preferred_element_type=jnp.float32)
        h = up * jax.nn.sigmoid(up) * scale
        h_ref[...] = h.astype(jnp.bfloat16)

    o_ref[...] = jax.lax.dot_general(
        h_ref[...], wd_ref[...].astype(jnp.bfloat16),
        (((1,), (1,)), ((), ())), preferred_element_type=jnp.float32)


@jax.jit
def kernel(x, W_gate, W_up, W_down):
    B_, S_, D_ = x.shape
    T = B_ * S_
    xf = x.reshape(T, D_)
    ne, ed = W_gate.shape[0], W_up.shape[0]

    out = pl.pallas_call(
        _moe_body,
        grid=(T // _TILE, D_ // _DCH),
        in_specs=[
            pl.BlockSpec((_TILE, D_), lambda i, j: (i, 0)),
            pl.BlockSpec((ne, D_), lambda i, j: (0, 0)),
            pl.BlockSpec((ed, D_), lambda i, j: (0, 0)),
            pl.BlockSpec((_DCH, ed), lambda i, j: (j, 0)),
        ],
        out_specs=pl.BlockSpec((_TILE, _DCH), lambda i, j: (i, j)),
        out_shape=jax.ShapeDtypeStruct((T, D_), jnp.float32),
        scratch_shapes=[pltpu.VMEM((_TILE, ed), jnp.bfloat16)],
        compiler_params=pltpu.CompilerParams(
            dimension_semantics=("parallel", "arbitrary")),
    )(xf, W_gate, W_up, W_down)
    return out.reshape(B_, S_, D_)


# speedup vs baseline: 1.5820x; 1.5820x over previous
import functools

import jax
import jax.numpy as jnp
from jax.experimental import pallas as pl
from jax.experimental.pallas import tpu as pltpu

_TILE = 2048


def _moe_body(x_ref, wg_ref, wu_ref, wd_ref, o_ref):
    xt = x_ref[...]

    logits = jax.lax.dot_general(
        xt, wg_ref[...], (((1,), (1,)), ((), ())),
        preferred_element_type=jnp.float32)
    ne = logits.shape[-1]
    m1 = jnp.max(logits, axis=-1, keepdims=True)
    e = jnp.exp(logits - m1)
    denom = jnp.sum(e, axis=-1, keepdims=True)
    col = jax.lax.broadcasted_iota(jnp.int32, logits.shape, 1)
    is_max = logits == m1
    argmax1 = jnp.min(jnp.where(is_max, col, ne), axis=-1, keepdims=True)
    e2 = jnp.max(jnp.where(col == argmax1, 0.0, e), axis=-1, keepdims=True)
    p = (1.0 + e2) / denom
    scale = p / (p + 1e-9)

    xb = xt.astype(jnp.bfloat16)
    up = jax.lax.dot_general(
        xb, wu_ref[...].astype(jnp.bfloat16), (((1,), (1,)), ((), ())),
        preferred_element_type=jnp.float32)
    h = (up * jax.nn.sigmoid(up)).astype(jnp.bfloat16)
    out = jax.lax.dot_general(
        h, wd_ref[...].astype(jnp.bfloat16), (((1,), (1,)), ((), ())),
        preferred_element_type=jnp.float32)
    o_ref[...] = out * scale


@jax.jit
def kernel(x, W_gate, W_up, W_down):
    B_, S_, D_ = x.shape
    T = B_ * S_
    xf = x.reshape(T, D_)
    ne, ed = W_gate.shape[0], W_up.shape[0]

    grid = (T // _TILE,)
    out = pl.pallas_call(
        _moe_body,
        grid=grid,
        in_specs=[
            pl.BlockSpec((_TILE, D_), lambda i: (i, 0)),
            pl.BlockSpec((ne, D_), lambda i: (0, 0)),
            pl.BlockSpec((ed, D_), lambda i: (0, 0)),
            pl.BlockSpec((D_, ed), lambda i: (0, 0)),
        ],
        out_specs=pl.BlockSpec((_TILE, D_), lambda i: (i, 0)),
        out_shape=jax.ShapeDtypeStruct((T, D_), jnp.float32),
        compiler_params=pltpu.CompilerParams(
            dimension_semantics=("parallel",)),
    )(xf, W_gate, W_up, W_down)
    return out.reshape(B_, S_, D_)
